# Initial kernel scaffold; baseline (speedup 1.0000x reference)
import jax
import jax.numpy as jnp
from jax.experimental import pallas as pl


def kernel(x, edge_index, W0, b0, W1a, b1a, g1, be1, W1b, b1b, W2a, b2a, g2, be2, W2b, b2b):
    raise NotImplementedError("write your pallas kernel here")


# speedup vs baseline: 4.4909x; 4.4909x over previous
import functools

import jax
import jax.numpy as jnp
from jax import lax
from jax.experimental import pallas as pl
from jax.experimental.pallas import tpu as pltpu
from jax.experimental.pallas import tpu_sc as plsc

N = 10000
D = 128
H = 128
E = 320000
SCALE = 20.0

NTILES = 32
C = 128
S = (E + NTILES * C - 1) // (NTILES * C)
EPAD = NTILES * S * C
NPAD = 10112
DUMMY = 10111
ROWS_Z = NPAD // 16

_PREC = jax.lax.Precision.DEFAULT


def _proj_body(x_ref, w_ref, b_ref, o_ref):
    o_ref[...] = (
        jnp.dot(x_ref[...], w_ref[...], preferred_element_type=jnp.float32,
                precision=_PREC)
        + b_ref[...]
    )


def _gin_mlp_body(h_ref, agg_ref, wa_ref, ba_ref, g_ref, be_ref, wb_ref,
                  bb_ref, o_ref, *, relu_out):
    h = h_ref[...]
    z = h + agg_ref[0, :N, :] + agg_ref[1, :N, :]
    z = jnp.dot(z, wa_ref[...], preferred_element_type=jnp.float32,
                precision=_PREC) + ba_ref[...]
    mu = jnp.mean(z, axis=0, keepdims=True)
    zc = z - mu
    var = jnp.mean(zc * zc, axis=0, keepdims=True)
    z = zc * lax.rsqrt(var + 1e-5) * g_ref[...] + be_ref[...]
    z = jnp.maximum(z, 0.0)
    z = jnp.dot(z, wb_ref[...], preferred_element_type=jnp.float32,
                precision=_PREC) + bb_ref[...]
    col_mean = jnp.mean(z, axis=0, keepdims=True)
    rownorm = jnp.sqrt(1e-6 + jnp.sum(z * z, axis=1, keepdims=True))
    z = SCALE * z / rownorm - col_mean
    if relu_out:
        z = jnp.maximum(z, 0.0)
    o_ref[...] = z


_proj = pl.pallas_call(
    _proj_body,
    out_shape=jax.ShapeDtypeStruct((N, H), jnp.float32),
)

_gin_mlp_relu = pl.pallas_call(
    functools.partial(_gin_mlp_body, relu_out=True),
    out_shape=jax.ShapeDtypeStruct((N, H), jnp.float32),
)

_gin_mlp_final = pl.pallas_call(
    functools.partial(_gin_mlp_body, relu_out=False),
    out_shape=jax.ShapeDtypeStruct((N, H), jnp.float32),
)


_sc_mesh = plsc.VectorSubcoreMesh(core_axis_name="c", subcore_axis_name="s")


@functools.partial(
    pl.kernel,
    out_type=jax.ShapeDtypeStruct((2, NPAD, H), jnp.float32),
    mesh=_sc_mesh,
    scratch_types=[
        pltpu.VMEM((S, C), jnp.int32),
        pltpu.VMEM((S, C), jnp.int32),
        pltpu.VMEM((C, H), jnp.float32),
        pltpu.VMEM_SHARED((NPAD, H), jnp.float32),
        pltpu.SemaphoreType.DMA,
    ],
)
def _gin_agg(h_hbm, src_hbm, dst_hbm, zero_hbm, out_hbm,
             src_v, dst_v, rows, agg, sem):
    cid = lax.axis_index("c")
    sid = lax.axis_index("s")
    wid = sid * 2 + cid

    pltpu.sync_copy(src_hbm.at[wid], src_v)
    pltpu.sync_copy(dst_hbm.at[wid], dst_v)
    pltpu.sync_copy(zero_hbm.at[pl.ds(sid * ROWS_Z, ROWS_Z)],
                    agg.at[pl.ds(sid * ROWS_Z, ROWS_Z)])
    plsc.subcore_barrier()

    def step(j, carry):
        pltpu.async_copy(h_hbm.at[src_v.at[j]], rows, sem).wait()
        pltpu.sync_copy(rows, agg.at[dst_v.at[j]], add=True)
        return carry

    lax.fori_loop(0, S, step, 0)

    plsc.subcore_barrier()
    pltpu.sync_copy(agg.at[pl.ds(sid * ROWS_Z, ROWS_Z)],
                    out_hbm.at[cid, pl.ds(sid * ROWS_Z, ROWS_Z)])


def kernel(x, edge_index, W0, b0, W1a, b1a, g1, be1, W1b, b1b,
           W2a, b2a, g2, be2, W2b, b2b):
    src = edge_index[0]
    dst = edge_index[1]
    pad = EPAD - E
    src_p = jnp.concatenate(
        [src, jnp.zeros((pad,), jnp.int32)]).reshape(NTILES, S, C)
    dst_p = jnp.concatenate(
        [dst, jnp.full((pad,), DUMMY, jnp.int32)]).reshape(NTILES, S, C)
    zeros = jnp.zeros((NPAD, H), jnp.float32)

    b0r = b0.reshape(1, H)
    b1ar = b1a.reshape(1, H)
    g1r = g1.reshape(1, H)
    be1r = be1.reshape(1, H)
    b1br = b1b.reshape(1, H)
    b2ar = b2a.reshape(1, H)
    g2r = g2.reshape(1, H)
    be2r = be2.reshape(1, H)
    b2br = b2b.reshape(1, H)

    h = _proj(x, W0, b0r)
    agg1 = _gin_agg(h, src_p, dst_p, zeros)
    h = _gin_mlp_relu(h, agg1, W1a, b1ar, g1r, be1r, W1b, b1br)
    agg2 = _gin_agg(h, src_p, dst_p, zeros)
    h = _gin_mlp_final(h, agg2, W2a, b2ar, g2r, be2r, W2b, b2br)
    return h
